# baseline (speedup 1.0000x reference)
import functools
import jax
import jax.numpy as jnp
from jax import lax
from jax.experimental import pallas as pl
from jax.experimental.pallas import tpu as pltpu
from jax.experimental.pallas import tpu_sc as plsc

N = 5000
NCH = 313
NPAD = NCH * 16
CAP = NPAD + 16
K_OUT = 100
SCORE_T = 0.05
NMS_T = 0.5
T0 = 0.97
TSTEP = 0.05
NEG = -1e9
NT = 16


def _body(boxes_hbm, scores_hbm, out_hbm,
          s_v, bx_v, cidx_v, cx1_v, cy1_v, cx2_v, cy2_v, cs_v, ca_v,
          ks_v, cnt_v, mx_v, out_v, ks_sh, cnt_sh):
    cid = lax.axis_index("c")
    sid = lax.axis_index("s")

    @pl.when(cid == 0)
    def _():
        iota = lax.iota(jnp.int32, 16)
        zeros_i = jnp.zeros((16,), jnp.int32)
        zeros_f = jnp.zeros((16,), jnp.float32)

        s_v[pl.ds(NPAD - 16, 16)] = zeros_f + jnp.float32(NEG)
        pltpu.sync_copy(scores_hbm, s_v.at[pl.ds(0, N)])
        pltpu.sync_copy(boxes_hbm, bx_v.at[pl.ds(0, 4 * N)])
        bx_v[pl.ds(4 * N, 16)] = zeros_f

        def compact_round(t):
            def compact_chunk(i, base_vec):
                sv = s_v[pl.ds(i * 16, 16)]
                m = sv > t

                @pl.when(jnp.any(m))
                def _():
                    mi = m.astype(jnp.int32)
                    pos = jnp.maximum(base_vec + plsc.cumsum(mi) - 1, 0)
                    plsc.store_scatter(cidx_v, [pos], i * 16 + iota, mask=m)

                return base_vec + plsc.all_reduce_population_count(m)

            base_vec = lax.fori_loop(0, NCH, compact_chunk, zeros_i)
            base_vec = lax.fori_loop(0, NCH, compact_chunk, zeros_i)
            k = jnp.max(base_vec)
            plsc.store_scatter(cidx_v, [k + iota], zeros_i + N)
            nck = (k + 15) // 16

            def gather_chunk(c, _):
                o = c * 16
                ci = cidx_v[pl.ds(o, 16)]
                c4 = ci * 4
                x1 = plsc.load_gather(bx_v, [c4])
                y1 = plsc.load_gather(bx_v, [c4 + 1])
                x2 = plsc.load_gather(bx_v, [c4 + 2])
                y2 = plsc.load_gather(bx_v, [c4 + 3])
                sc = plsc.load_gather(s_v, [ci])
                cx1_v[pl.ds(o, 16)] = x1
                cy1_v[pl.ds(o, 16)] = y1
                cx2_v[pl.ds(o, 16)] = x2
                cy2_v[pl.ds(o, 16)] = y2
                cs_v[pl.ds(o, 16)] = sc
                ca_v[pl.ds(o, 16)] = (x2 - x1) * (y2 - y1)
                return 0

            lax.fori_loop(0, nck, gather_chunk, 0)

            kp = nck * 16

            def supp_chunk(z, nsurv):
                c = sid + z * NT
                o = c * 16
                x1 = cx1_v[pl.ds(o, 16)]
                y1 = cy1_v[pl.ds(o, 16)]
                x2 = cx2_v[pl.ds(o, 16)]
                y2 = cy2_v[pl.ds(o, 16)]
                si = cs_v[pl.ds(o, 16)]
                ai = ca_v[pl.ds(o, 16)]
                ii = cidx_v[pl.ds(o, 16)]

                def jbody(j, sup):
                    jb = zeros_i + j
                    xj1 = plsc.load_gather(cx1_v, [jb])
                    yj1 = plsc.load_gather(cy1_v, [jb])
                    xj2 = plsc.load_gather(cx2_v, [jb])
                    yj2 = plsc.load_gather(cy2_v, [jb])
                    sj = plsc.load_gather(cs_v, [jb])
                    aj = plsc.load_gather(ca_v, [jb])
                    ij = plsc.load_gather(cidx_v, [jb])
                    w = jnp.maximum(jnp.minimum(x2, xj2) - jnp.maximum(x1, xj1), 0.0)
                    h = jnp.maximum(jnp.minimum(y2, yj2) - jnp.maximum(y1, yj1), 0.0)
                    inter = w * h
                    iou = inter / (aj + ai - inter + 1e-9)
                    prio = (sj > si) | ((sj == si) & (ij < ii))
                    return sup | ((iou > NMS_T) & prio)

                sup = lax.fori_loop(0, kp, jbody, jnp.zeros((16,), jnp.bool_))
                keep = jnp.logical_and(jnp.logical_not(sup), si > SCORE_T)
                ks = jnp.where(keep, si, jnp.float32(NEG))
                ks_v[pl.ds(o, 16)] = ks
                pltpu.sync_copy(ks_v.at[pl.ds(o, 16)], ks_sh.at[pl.ds(o, 16)])
                return nsurv + jnp.sum(keep.astype(jnp.int32))

            nz = jnp.maximum(0, (nck - sid + NT - 1) // NT)
            nsurv_loc = lax.fori_loop(0, nz, supp_chunk, jnp.int32(0))

            cnt_v[pl.ds(0, 16)] = zeros_i + nsurv_loc
            pltpu.sync_copy(cnt_v.at[pl.ds(0, 16)], cnt_sh.at[pl.ds(sid * 16, 16)])
            plsc.subcore_barrier()
            pltpu.sync_copy(cnt_sh, cnt_v.at[pl.ds(16, NT * 16)])

            def sum_counts(tt, acc):
                return acc + cnt_v[pl.ds(16 + tt * 16, 16)]

            nsurv = jnp.max(lax.fori_loop(0, NT, sum_counts, zeros_i))
            plsc.subcore_barrier()
            return k, nck, nsurv

        def cond(carry):
            t_next, t_used, nck, nsurv = carry
            return jnp.logical_and(nsurv < K_OUT, t_used > SCORE_T)

        def round_fn(carry):
            t_next, _, _, _ = carry
            _, nck, nsurv = compact_round(t_next)
            nt = jnp.maximum(t_next - jnp.float32(TSTEP), jnp.float32(SCORE_T))
            return nt, t_next, nck, nsurv

        init = (jnp.float32(T0), jnp.float32(2.0), jnp.int32(0), jnp.int32(-1))
        _, _, nck, _ = lax.while_loop(cond, round_fn, init)

        @pl.when(sid == 0)
        def _():
            pltpu.sync_copy(ks_sh, ks_v)
            iota4 = jnp.minimum(iota, 3)

            def _emit_general():

                ngrp = (nck + 15) // 16

                def init_grp(g, _):
                    def init_chunk(u, mx):
                        c = g * 16 + u
                        inb = c < nck
                        cm = jnp.where(
                            inb, jnp.max(ks_v[pl.ds(jnp.minimum(c, CAP // 16 - 1) * 16, 16)]),
                            jnp.float32(NEG))
                        return jnp.where(iota == u, cm, mx)

                    mx = lax.fori_loop(0, 16, init_chunk, zeros_f + jnp.float32(NEG))
                    mx_v[pl.ds(g * 16, 16)] = mx
                    return 0

                lax.fori_loop(0, ngrp, init_grp, 0)

                def emit_row(r, _):
                    def gmax(g, mm):
                        return jnp.maximum(mm, jnp.max(mx_v[pl.ds(g * 16, 16)]))

                    m = lax.fori_loop(0, ngrp, gmax, jnp.float32(NEG))
                    valid = m > jnp.float32(-1e8)

                    def gsel(g, gs):
                        mx = mx_v[pl.ds(g * 16, 16)]
                        hit = jnp.max(mx) == m
                        return jnp.minimum(gs, jnp.where(hit, g, jnp.int32(1 << 20)))

                    gstar = lax.fori_loop(0, ngrp, gsel, jnp.int32(1 << 20))
                    gstar = jnp.minimum(gstar, ngrp - 1)
                    mx = mx_v[pl.ds(gstar * 16, 16)]
                    ustar = jnp.min(jnp.where(mx == m, iota, jnp.int32(1 << 20)))
                    cstar = jnp.minimum(gstar * 16 + ustar, CAP // 16 - 1)

                    o = cstar * 16
                    v = ks_v[pl.ds(o, 16)]
                    ci = cidx_v[pl.ds(o, 16)]
                    sel = jnp.min(jnp.where(v == m, ci, jnp.int32(1 << 30)))
                    hit = jnp.logical_and(v == m, ci == sel)
                    nv = jnp.where(hit, jnp.float32(NEG), v)
                    ks_v[pl.ds(o, 16)] = nv
                    newm = jnp.max(nv)
                    grp = jnp.where(iota == ustar, newm, mx_v[pl.ds(gstar * 16, 16)])
                    mx_v[pl.ds(gstar * 16, 16)] = grp

                    src = jnp.where(valid, sel, jnp.int32(N))
                    coords = plsc.load_gather(bx_v, [src * 4 + iota4])
                    mscore = jnp.where(valid, m, jnp.float32(0.0))
                    row = jnp.where(iota < 4, coords,
                                    jnp.where(iota == 4, mscore, 0.0))
                    plsc.store_scatter(out_v, [r * 5 + iota], row, mask=iota < 5)
                    return 0

                lax.fori_loop(0, K_OUT, emit_row, 0)

            @pl.when(nck <= 16)
            def _():
                def initmax(c, mx):
                    cm = jnp.max(ks_v[pl.ds(c * 16, 16)])
                    return jnp.where(iota == c, cm, mx)

                maxes0 = lax.fori_loop(0, nck, initmax,
                                       zeros_f + jnp.float32(NEG))

                def emit_row_fast(r, maxes):
                    m = jnp.max(maxes)
                    valid = m > jnp.float32(-1e8)
                    cstar = plsc.all_reduce_ffs(maxes == m)
                    pos = cstar * 16 + iota
                    v = plsc.load_gather(ks_v, [pos])
                    ci = plsc.load_gather(cidx_v, [pos])
                    lstar = plsc.all_reduce_ffs(v == m)
                    sel = plsc.load_gather(cidx_v, [cstar * 16 + lstar])
                    nv = jnp.where(iota == lstar, jnp.float32(NEG), v)
                    plsc.store_scatter(ks_v, [pos], nv)
                    newm = jnp.max(nv)
                    maxes = jnp.where(iota == cstar, newm, maxes)
                    src = jnp.where(valid, sel, jnp.int32(N))
                    coords = plsc.load_gather(bx_v, [src * 4 + iota4])
                    mscore = jnp.where(valid, m, jnp.float32(0.0))
                    row = jnp.where(iota < 4, coords,
                                    jnp.where(iota == 4, mscore, 0.0))
                    plsc.store_scatter(out_v, [r * 5 + iota], row,
                                       mask=iota < 5)
                    return maxes

                lax.fori_loop(0, K_OUT, emit_row_fast, maxes0)

            @pl.when(nck > 16)
            def _general_path():
                _emit_general()

            pltpu.sync_copy(out_v, out_hbm)


@jax.jit
def _nms(boxes, scores):
    f = functools.partial(
        pl.kernel,
        out_type=jax.ShapeDtypeStruct((512,), jnp.float32),
        mesh=plsc.VectorSubcoreMesh(
            core_axis_name="c", subcore_axis_name="s",
            num_cores=1, num_subcores=16),
        compiler_params=pltpu.CompilerParams(needs_layout_passes=False),
        scratch_types=[
            pltpu.VMEM((NPAD,), jnp.float32),
            pltpu.VMEM((4 * N + 64,), jnp.float32),
            pltpu.VMEM((CAP,), jnp.int32),
            pltpu.VMEM((CAP,), jnp.float32),
            pltpu.VMEM((CAP,), jnp.float32),
            pltpu.VMEM((CAP,), jnp.float32),
            pltpu.VMEM((CAP,), jnp.float32),
            pltpu.VMEM((CAP,), jnp.float32),
            pltpu.VMEM((CAP,), jnp.float32),
            pltpu.VMEM((CAP,), jnp.float32),
            pltpu.VMEM((16 + NT * 16,), jnp.int32),
            pltpu.VMEM((352,), jnp.float32),
            pltpu.VMEM((512,), jnp.float32),
            pltpu.VMEM_SHARED((CAP,), jnp.float32),
            pltpu.VMEM_SHARED((NT * 16,), jnp.int32),
        ],
    )(_body)
    return f(boxes, scores)


def kernel(boxes, scores):
    out = _nms(boxes.reshape(-1), scores)
    return out[:500].reshape(K_OUT, 5)


# speedup vs baseline: 1.4128x; 1.4128x over previous
import functools
import jax
import jax.numpy as jnp
from jax import lax
from jax.experimental import pallas as pl
from jax.experimental.pallas import tpu as pltpu
from jax.experimental.pallas import tpu_sc as plsc

N = 5000
NCH = 313
NPAD = NCH * 16
CAP = NPAD + 16
K_OUT = 100
SCORE_T = 0.05
NMS_T = 0.5
T0 = 0.97
TSTEP = 0.05
NEG = -1e9
NT = 16


def _body(boxes_hbm, scores_hbm, out_hbm,
          s_v, bx_v, cidx_v, cx1_v, cy1_v, cx2_v, cy2_v, cs_v, ca_v,
          ks_v, cnt_v, mx_v, out_v, ks_sh, cnt_sh):
    cid = lax.axis_index("c")
    sid = lax.axis_index("s")

    @pl.when(cid == 0)
    def _():
        iota = lax.iota(jnp.int32, 16)
        zeros_i = jnp.zeros((16,), jnp.int32)
        zeros_f = jnp.zeros((16,), jnp.float32)

        s_v[pl.ds(NPAD - 16, 16)] = zeros_f + jnp.float32(NEG)
        pltpu.sync_copy(scores_hbm, s_v.at[pl.ds(0, N)])
        pltpu.sync_copy(boxes_hbm, bx_v.at[pl.ds(0, 4 * N)])
        bx_v[pl.ds(4 * N, 16)] = zeros_f

        def compact_round(t):
            def compact_chunk(i, base_vec):
                sv = s_v[pl.ds(i * 16, 16)]
                m = sv > t
                mi = m.astype(jnp.int32)
                pos = jnp.maximum(base_vec + plsc.cumsum(mi) - 1, 0)
                plsc.store_scatter(cidx_v, [pos], i * 16 + iota, mask=m)
                return base_vec + plsc.all_reduce_population_count(m)

            base_vec = lax.fori_loop(0, NCH, compact_chunk, zeros_i)
            k = jnp.max(base_vec)
            plsc.store_scatter(cidx_v, [k + iota], zeros_i + N)
            nck = (k + 15) // 16

            def gather_chunk(c, _):
                o = c * 16
                ci = cidx_v[pl.ds(o, 16)]
                c4 = ci * 4
                x1 = plsc.load_gather(bx_v, [c4])
                y1 = plsc.load_gather(bx_v, [c4 + 1])
                x2 = plsc.load_gather(bx_v, [c4 + 2])
                y2 = plsc.load_gather(bx_v, [c4 + 3])
                sc = plsc.load_gather(s_v, [ci])
                cx1_v[pl.ds(o, 16)] = x1
                cy1_v[pl.ds(o, 16)] = y1
                cx2_v[pl.ds(o, 16)] = x2
                cy2_v[pl.ds(o, 16)] = y2
                cs_v[pl.ds(o, 16)] = sc
                ca_v[pl.ds(o, 16)] = (x2 - x1) * (y2 - y1)
                return 0

            lax.fori_loop(0, nck, gather_chunk, 0)

            kp = nck * 16

            def supp_chunk(z, nsurv):
                c = sid + z * NT
                o = c * 16
                x1 = cx1_v[pl.ds(o, 16)]
                y1 = cy1_v[pl.ds(o, 16)]
                x2 = cx2_v[pl.ds(o, 16)]
                y2 = cy2_v[pl.ds(o, 16)]
                si = cs_v[pl.ds(o, 16)]
                ai = ca_v[pl.ds(o, 16)]
                ii = cidx_v[pl.ds(o, 16)]

                def jbody(j, sup):
                    jb = zeros_i + j
                    xj1 = plsc.load_gather(cx1_v, [jb])
                    yj1 = plsc.load_gather(cy1_v, [jb])
                    xj2 = plsc.load_gather(cx2_v, [jb])
                    yj2 = plsc.load_gather(cy2_v, [jb])
                    sj = plsc.load_gather(cs_v, [jb])
                    aj = plsc.load_gather(ca_v, [jb])
                    ij = plsc.load_gather(cidx_v, [jb])
                    w = jnp.maximum(jnp.minimum(x2, xj2) - jnp.maximum(x1, xj1), 0.0)
                    h = jnp.maximum(jnp.minimum(y2, yj2) - jnp.maximum(y1, yj1), 0.0)
                    inter = w * h
                    iou = inter / (aj + ai - inter + 1e-9)
                    prio = (sj > si) | ((sj == si) & (ij < ii))
                    return sup | ((iou > NMS_T) & prio)

                sup = lax.fori_loop(0, kp, jbody, jnp.zeros((16,), jnp.bool_))
                keep = jnp.logical_and(jnp.logical_not(sup), si > SCORE_T)
                ks = jnp.where(keep, si, jnp.float32(NEG))
                ks_v[pl.ds(o, 16)] = ks
                pltpu.sync_copy(ks_v.at[pl.ds(o, 16)], ks_sh.at[pl.ds(o, 16)])
                return nsurv + jnp.sum(keep.astype(jnp.int32))

            nz = jnp.maximum(0, (nck - sid + NT - 1) // NT)
            nsurv_loc = lax.fori_loop(0, nz, supp_chunk, jnp.int32(0))

            cnt_v[pl.ds(0, 16)] = zeros_i + nsurv_loc
            pltpu.sync_copy(cnt_v.at[pl.ds(0, 16)], cnt_sh.at[pl.ds(sid * 16, 16)])
            plsc.subcore_barrier()
            pltpu.sync_copy(cnt_sh, cnt_v.at[pl.ds(16, NT * 16)])

            def sum_counts(tt, acc):
                return acc + cnt_v[pl.ds(16 + tt * 16, 16)]

            nsurv = jnp.max(lax.fori_loop(0, NT, sum_counts, zeros_i))
            plsc.subcore_barrier()
            return k, nck, nsurv

        def cond(carry):
            t_next, t_used, nck, nsurv = carry
            return jnp.logical_and(nsurv < K_OUT, t_used > SCORE_T)

        def round_fn(carry):
            t_next, _, _, _ = carry
            _, nck, nsurv = compact_round(t_next)
            nt = jnp.maximum(t_next - jnp.float32(TSTEP), jnp.float32(SCORE_T))
            return nt, t_next, nck, nsurv

        init = (jnp.float32(T0), jnp.float32(2.0), jnp.int32(0), jnp.int32(-1))
        _, _, nck, _ = lax.while_loop(cond, round_fn, init)

        @pl.when(sid == 0)
        def _():
            pltpu.sync_copy(ks_sh, ks_v)
            iota4 = jnp.minimum(iota, 3)

            def _emit_general():

                ngrp = (nck + 15) // 16

                def init_grp(g, _):
                    def init_chunk(u, mx):
                        c = g * 16 + u
                        inb = c < nck
                        cm = jnp.where(
                            inb, jnp.max(ks_v[pl.ds(jnp.minimum(c, CAP // 16 - 1) * 16, 16)]),
                            jnp.float32(NEG))
                        return jnp.where(iota == u, cm, mx)

                    mx = lax.fori_loop(0, 16, init_chunk, zeros_f + jnp.float32(NEG))
                    mx_v[pl.ds(g * 16, 16)] = mx
                    return 0

                lax.fori_loop(0, ngrp, init_grp, 0)

                def emit_row(r, _):
                    def gmax(g, mm):
                        return jnp.maximum(mm, jnp.max(mx_v[pl.ds(g * 16, 16)]))

                    m = lax.fori_loop(0, ngrp, gmax, jnp.float32(NEG))
                    valid = m > jnp.float32(-1e8)

                    def gsel(g, gs):
                        mx = mx_v[pl.ds(g * 16, 16)]
                        hit = jnp.max(mx) == m
                        return jnp.minimum(gs, jnp.where(hit, g, jnp.int32(1 << 20)))

                    gstar = lax.fori_loop(0, ngrp, gsel, jnp.int32(1 << 20))
                    gstar = jnp.minimum(gstar, ngrp - 1)
                    mx = mx_v[pl.ds(gstar * 16, 16)]
                    ustar = jnp.min(jnp.where(mx == m, iota, jnp.int32(1 << 20)))
                    cstar = jnp.minimum(gstar * 16 + ustar, CAP // 16 - 1)

                    o = cstar * 16
                    v = ks_v[pl.ds(o, 16)]
                    ci = cidx_v[pl.ds(o, 16)]
                    sel = jnp.min(jnp.where(v == m, ci, jnp.int32(1 << 30)))
                    hit = jnp.logical_and(v == m, ci == sel)
                    nv = jnp.where(hit, jnp.float32(NEG), v)
                    ks_v[pl.ds(o, 16)] = nv
                    newm = jnp.max(nv)
                    grp = jnp.where(iota == ustar, newm, mx_v[pl.ds(gstar * 16, 16)])
                    mx_v[pl.ds(gstar * 16, 16)] = grp

                    src = jnp.where(valid, sel, jnp.int32(N))
                    coords = plsc.load_gather(bx_v, [src * 4 + iota4])
                    mscore = jnp.where(valid, m, jnp.float32(0.0))
                    row = jnp.where(iota < 4, coords,
                                    jnp.where(iota == 4, mscore, 0.0))
                    plsc.store_scatter(out_v, [r * 5 + iota], row, mask=iota < 5)
                    return 0

                lax.fori_loop(0, K_OUT, emit_row, 0)

            @pl.when(nck <= 16)
            def _():
                def initmax(c, mx):
                    cm = jnp.max(ks_v[pl.ds(c * 16, 16)])
                    return jnp.where(iota == c, cm, mx)

                maxes0 = lax.fori_loop(0, nck, initmax,
                                       zeros_f + jnp.float32(NEG))

                def emit_row_fast(r, maxes):
                    m = jnp.max(maxes)
                    valid = m > jnp.float32(-1e8)
                    cstar = plsc.all_reduce_ffs(maxes == m)
                    pos = cstar * 16 + iota
                    v = plsc.load_gather(ks_v, [pos])
                    ci = plsc.load_gather(cidx_v, [pos])
                    lstar = plsc.all_reduce_ffs(v == m)
                    sel = plsc.load_gather(cidx_v, [cstar * 16 + lstar])
                    nv = jnp.where(iota == lstar, jnp.float32(NEG), v)
                    plsc.store_scatter(ks_v, [pos], nv)
                    newm = jnp.max(nv)
                    maxes = jnp.where(iota == cstar, newm, maxes)
                    src = jnp.where(valid, sel, jnp.int32(N))
                    coords = plsc.load_gather(bx_v, [src * 4 + iota4])
                    mscore = jnp.where(valid, m, jnp.float32(0.0))
                    row = jnp.where(iota < 4, coords,
                                    jnp.where(iota == 4, mscore, 0.0))
                    plsc.store_scatter(out_v, [r * 5 + iota], row,
                                       mask=iota < 5)
                    return maxes

                lax.fori_loop(0, K_OUT, emit_row_fast, maxes0)

            @pl.when(nck > 16)
            def _general_path():
                _emit_general()

            pltpu.sync_copy(out_v, out_hbm)


@jax.jit
def _nms(boxes, scores):
    f = functools.partial(
        pl.kernel,
        out_type=jax.ShapeDtypeStruct((512,), jnp.float32),
        mesh=plsc.VectorSubcoreMesh(
            core_axis_name="c", subcore_axis_name="s",
            num_cores=1, num_subcores=16),
        compiler_params=pltpu.CompilerParams(needs_layout_passes=False),
        scratch_types=[
            pltpu.VMEM((NPAD,), jnp.float32),
            pltpu.VMEM((4 * N + 64,), jnp.float32),
            pltpu.VMEM((CAP,), jnp.int32),
            pltpu.VMEM((CAP,), jnp.float32),
            pltpu.VMEM((CAP,), jnp.float32),
            pltpu.VMEM((CAP,), jnp.float32),
            pltpu.VMEM((CAP,), jnp.float32),
            pltpu.VMEM((CAP,), jnp.float32),
            pltpu.VMEM((CAP,), jnp.float32),
            pltpu.VMEM((CAP,), jnp.float32),
            pltpu.VMEM((16 + NT * 16,), jnp.int32),
            pltpu.VMEM((352,), jnp.float32),
            pltpu.VMEM((512,), jnp.float32),
            pltpu.VMEM_SHARED((CAP,), jnp.float32),
            pltpu.VMEM_SHARED((NT * 16,), jnp.int32),
        ],
    )(_body)
    return f(boxes, scores)


def kernel(boxes, scores):
    out = _nms(boxes.reshape(-1), scores)
    return out[:500].reshape(K_OUT, 5)


# speedup vs baseline: 1.5409x; 1.0907x over previous
import functools
import jax
import jax.numpy as jnp
from jax import lax
from jax.experimental import pallas as pl
from jax.experimental.pallas import tpu as pltpu
from jax.experimental.pallas import tpu_sc as plsc

N = 5000
NCH = 313
NPAD = NCH * 16
CAP = NPAD + 16
K_OUT = 100
SCORE_T = 0.05
NMS_T = 0.5
T0 = 0.97
TSTEP = 0.05
NEG = -1e9
NT = 16
CPT = 20
SLOT = 336


def _body(boxes_hbm, scores_hbm, out_hbm,
          s_v, bx_v, cidx_v, cx1_v, cy1_v, cx2_v, cy2_v, cs_v, ca_v,
          ks_v, cnt_v, lidx_v, lall_v, pfx_v, mx_v, out_v, ks_sh, cnt_sh,
          idx_sh):
    cid = lax.axis_index("c")
    sid = lax.axis_index("s")

    @pl.when(cid == 0)
    def _():
        iota = lax.iota(jnp.int32, 16)
        zeros_i = jnp.zeros((16,), jnp.int32)
        zeros_f = jnp.zeros((16,), jnp.float32)

        s_v[pl.ds(NPAD - 16, 16)] = zeros_f + jnp.float32(NEG)
        pltpu.sync_copy(scores_hbm, s_v.at[pl.ds(0, N)])
        pltpu.sync_copy(boxes_hbm, bx_v.at[pl.ds(0, 4 * N)])
        bx_v[pl.ds(4 * N, 16)] = zeros_f

        def compact_round(t):
            def compact_chunk(i, base_vec):
                sv = s_v[pl.ds(i * 16, 16)]
                m = sv > t
                mi = m.astype(jnp.int32)
                pos = jnp.maximum(base_vec + plsc.cumsum(mi) - 1, 0)
                plsc.store_scatter(lidx_v, [pos], i * 16 + iota, mask=m)
                return base_vec + plsc.all_reduce_population_count(m)

            lo = sid * CPT
            hi = jnp.minimum(jnp.int32(NCH), lo + CPT)
            base_vec = lax.fori_loop(lo, hi, compact_chunk, zeros_i)
            pltpu.sync_copy(lidx_v, idx_sh.at[pl.ds(sid * SLOT, SLOT)])
            cnt_v[pl.ds(0, 16)] = base_vec
            pltpu.sync_copy(cnt_v.at[pl.ds(0, 16)],
                            cnt_sh.at[pl.ds(sid * 16, 16)])
            plsc.subcore_barrier()
            pltpu.sync_copy(idx_sh, lall_v)
            pltpu.sync_copy(cnt_sh.at[pl.ds(0, NT * 16)],
                            cnt_v.at[pl.ds(16, NT * 16)])

            def getc(tt, cv):
                return jnp.where(iota == tt, cnt_v[pl.ds(16 + tt * 16, 16)], cv)

            cvec = lax.fori_loop(0, NT, getc, zeros_i)
            incl = plsc.cumsum(cvec)
            k = jnp.max(incl)
            pfx_v[pl.ds(0, 16)] = incl - cvec

            def asm_tile(tt, _):
                psplat = plsc.load_gather(pfx_v, [zeros_i + tt])
                ct = jnp.max(jnp.where(iota == tt, cvec, 0))

                def asm_chunk(u, _2):
                    vals = lall_v[pl.ds(tt * SLOT + u * 16, 16)]
                    ppos = jnp.minimum(psplat + u * 16 + iota,
                                       jnp.int32(CAP - 1))
                    plsc.store_scatter(cidx_v, [ppos], vals,
                                       mask=(u * 16 + iota) < ct)
                    return 0

                lax.fori_loop(0, (ct + 15) // 16, asm_chunk, 0)
                return 0

            lax.fori_loop(0, NT, asm_tile, 0)
            plsc.store_scatter(cidx_v, [k + iota], zeros_i + N)
            nck = (k + 15) // 16

            def gather_chunk(c, _):
                o = c * 16
                ci = cidx_v[pl.ds(o, 16)]
                c4 = ci * 4
                x1 = plsc.load_gather(bx_v, [c4])
                y1 = plsc.load_gather(bx_v, [c4 + 1])
                x2 = plsc.load_gather(bx_v, [c4 + 2])
                y2 = plsc.load_gather(bx_v, [c4 + 3])
                sc = plsc.load_gather(s_v, [ci])
                cx1_v[pl.ds(o, 16)] = x1
                cy1_v[pl.ds(o, 16)] = y1
                cx2_v[pl.ds(o, 16)] = x2
                cy2_v[pl.ds(o, 16)] = y2
                cs_v[pl.ds(o, 16)] = sc
                ca_v[pl.ds(o, 16)] = (x2 - x1) * (y2 - y1)
                return 0

            lax.fori_loop(0, nck, gather_chunk, 0)

            kp = nck * 16

            def supp_chunk(z, nsurv):
                c = sid + z * NT
                o = c * 16
                x1 = cx1_v[pl.ds(o, 16)]
                y1 = cy1_v[pl.ds(o, 16)]
                x2 = cx2_v[pl.ds(o, 16)]
                y2 = cy2_v[pl.ds(o, 16)]
                si = cs_v[pl.ds(o, 16)]
                ai = ca_v[pl.ds(o, 16)]
                ii = cidx_v[pl.ds(o, 16)]

                def jbody(j, sup):
                    jb = zeros_i + j
                    xj1 = plsc.load_gather(cx1_v, [jb])
                    yj1 = plsc.load_gather(cy1_v, [jb])
                    xj2 = plsc.load_gather(cx2_v, [jb])
                    yj2 = plsc.load_gather(cy2_v, [jb])
                    sj = plsc.load_gather(cs_v, [jb])
                    aj = plsc.load_gather(ca_v, [jb])
                    ij = plsc.load_gather(cidx_v, [jb])
                    w = jnp.maximum(jnp.minimum(x2, xj2) - jnp.maximum(x1, xj1), 0.0)
                    h = jnp.maximum(jnp.minimum(y2, yj2) - jnp.maximum(y1, yj1), 0.0)
                    inter = w * h
                    iou = inter / (aj + ai - inter + 1e-9)
                    prio = (sj > si) | ((sj == si) & (ij < ii))
                    return sup | ((iou > NMS_T) & prio)

                sup = lax.fori_loop(0, kp, jbody, jnp.zeros((16,), jnp.bool_))
                keep = jnp.logical_and(jnp.logical_not(sup), si > SCORE_T)
                ks = jnp.where(keep, si, jnp.float32(NEG))
                ks_v[pl.ds(o, 16)] = ks
                pltpu.sync_copy(ks_v.at[pl.ds(o, 16)], ks_sh.at[pl.ds(o, 16)])
                return nsurv + jnp.sum(keep.astype(jnp.int32))

            nz = jnp.maximum(0, (nck - sid + NT - 1) // NT)
            nsurv_loc = lax.fori_loop(0, nz, supp_chunk, jnp.int32(0))

            cnt_v[pl.ds(0, 16)] = zeros_i + nsurv_loc
            pltpu.sync_copy(cnt_v.at[pl.ds(0, 16)],
                            cnt_sh.at[pl.ds(256 + sid * 16, 16)])
            plsc.subcore_barrier()
            pltpu.sync_copy(cnt_sh.at[pl.ds(256, NT * 16)],
                            cnt_v.at[pl.ds(16, NT * 16)])

            def sum_counts(tt, acc):
                return acc + cnt_v[pl.ds(16 + tt * 16, 16)]

            nsurv = jnp.max(lax.fori_loop(0, NT, sum_counts, zeros_i))
            plsc.subcore_barrier()
            return k, nck, nsurv

        def cond(carry):
            t_next, t_used, nck, nsurv = carry
            return jnp.logical_and(nsurv < K_OUT, t_used > SCORE_T)

        def round_fn(carry):
            t_next, _, _, _ = carry
            _, nck, nsurv = compact_round(t_next)
            nt = jnp.maximum(t_next - jnp.float32(TSTEP), jnp.float32(SCORE_T))
            return nt, t_next, nck, nsurv

        init = (jnp.float32(T0), jnp.float32(2.0), jnp.int32(0), jnp.int32(-1))
        _, _, nck, _ = lax.while_loop(cond, round_fn, init)

        @pl.when(sid == 0)
        def _():
            pltpu.sync_copy(ks_sh, ks_v)
            iota4 = jnp.minimum(iota, 3)

            def _emit_general():

                ngrp = (nck + 15) // 16

                def init_grp(g, _):
                    def init_chunk(u, mx):
                        c = g * 16 + u
                        inb = c < nck
                        cm = jnp.where(
                            inb, jnp.max(ks_v[pl.ds(jnp.minimum(c, CAP // 16 - 1) * 16, 16)]),
                            jnp.float32(NEG))
                        return jnp.where(iota == u, cm, mx)

                    mx = lax.fori_loop(0, 16, init_chunk, zeros_f + jnp.float32(NEG))
                    mx_v[pl.ds(g * 16, 16)] = mx
                    return 0

                lax.fori_loop(0, ngrp, init_grp, 0)

                def emit_row(r, _):
                    def gmax(g, mm):
                        return jnp.maximum(mm, jnp.max(mx_v[pl.ds(g * 16, 16)]))

                    m = lax.fori_loop(0, ngrp, gmax, jnp.float32(NEG))
                    valid = m > jnp.float32(-1e8)

                    def gsel(g, gs):
                        mx = mx_v[pl.ds(g * 16, 16)]
                        hit = jnp.max(mx) == m
                        return jnp.minimum(gs, jnp.where(hit, g, jnp.int32(1 << 20)))

                    gstar = lax.fori_loop(0, ngrp, gsel, jnp.int32(1 << 20))
                    gstar = jnp.minimum(gstar, ngrp - 1)
                    mx = mx_v[pl.ds(gstar * 16, 16)]
                    ustar = jnp.min(jnp.where(mx == m, iota, jnp.int32(1 << 20)))
                    cstar = jnp.minimum(gstar * 16 + ustar, CAP // 16 - 1)

                    o = cstar * 16
                    v = ks_v[pl.ds(o, 16)]
                    ci = cidx_v[pl.ds(o, 16)]
                    sel = jnp.min(jnp.where(v == m, ci, jnp.int32(1 << 30)))
                    hit = jnp.logical_and(v == m, ci == sel)
                    nv = jnp.where(hit, jnp.float32(NEG), v)
                    ks_v[pl.ds(o, 16)] = nv
                    newm = jnp.max(nv)
                    grp = jnp.where(iota == ustar, newm, mx_v[pl.ds(gstar * 16, 16)])
                    mx_v[pl.ds(gstar * 16, 16)] = grp

                    src = jnp.where(valid, sel, jnp.int32(N))
                    coords = plsc.load_gather(bx_v, [src * 4 + iota4])
                    mscore = jnp.where(valid, m, jnp.float32(0.0))
                    row = jnp.where(iota < 4, coords,
                                    jnp.where(iota == 4, mscore, 0.0))
                    plsc.store_scatter(out_v, [r * 5 + iota], row, mask=iota < 5)
                    return 0

                lax.fori_loop(0, K_OUT, emit_row, 0)

            @pl.when(nck <= 16)
            def _():
                def initmax(c, mx):
                    cm = jnp.max(ks_v[pl.ds(c * 16, 16)])
                    return jnp.where(iota == c, cm, mx)

                maxes0 = lax.fori_loop(0, nck, initmax,
                                       zeros_f + jnp.float32(NEG))

                def emit_row_fast(r, maxes):
                    m = jnp.max(maxes)
                    valid = m > jnp.float32(-1e8)
                    cstar = plsc.all_reduce_ffs(maxes == m)
                    pos = cstar * 16 + iota
                    v = plsc.load_gather(ks_v, [pos])
                    ci = plsc.load_gather(cidx_v, [pos])
                    lstar = plsc.all_reduce_ffs(v == m)
                    sel = plsc.load_gather(cidx_v, [cstar * 16 + lstar])
                    nv = jnp.where(iota == lstar, jnp.float32(NEG), v)
                    plsc.store_scatter(ks_v, [pos], nv)
                    newm = jnp.max(nv)
                    maxes = jnp.where(iota == cstar, newm, maxes)
                    src = jnp.where(valid, sel, jnp.int32(N))
                    coords = plsc.load_gather(bx_v, [src * 4 + iota4])
                    mscore = jnp.where(valid, m, jnp.float32(0.0))
                    row = jnp.where(iota < 4, coords,
                                    jnp.where(iota == 4, mscore, 0.0))
                    plsc.store_scatter(out_v, [r * 5 + iota], row,
                                       mask=iota < 5)
                    return maxes

                lax.fori_loop(0, K_OUT, emit_row_fast, maxes0)

            @pl.when(nck > 16)
            def _general_path():
                _emit_general()

            pltpu.sync_copy(out_v, out_hbm)


@jax.jit
def _nms(boxes, scores):
    f = functools.partial(
        pl.kernel,
        out_type=jax.ShapeDtypeStruct((512,), jnp.float32),
        mesh=plsc.VectorSubcoreMesh(
            core_axis_name="c", subcore_axis_name="s",
            num_cores=1, num_subcores=16),
        compiler_params=pltpu.CompilerParams(needs_layout_passes=False),
        scratch_types=[
            pltpu.VMEM((NPAD,), jnp.float32),
            pltpu.VMEM((4 * N + 64,), jnp.float32),
            pltpu.VMEM((CAP,), jnp.int32),
            pltpu.VMEM((CAP,), jnp.float32),
            pltpu.VMEM((CAP,), jnp.float32),
            pltpu.VMEM((CAP,), jnp.float32),
            pltpu.VMEM((CAP,), jnp.float32),
            pltpu.VMEM((CAP,), jnp.float32),
            pltpu.VMEM((CAP,), jnp.float32),
            pltpu.VMEM((CAP,), jnp.float32),
            pltpu.VMEM((16 + NT * 16,), jnp.int32),
            pltpu.VMEM((SLOT,), jnp.int32),
            pltpu.VMEM((NT * SLOT,), jnp.int32),
            pltpu.VMEM((16,), jnp.int32),
            pltpu.VMEM((352,), jnp.float32),
            pltpu.VMEM((512,), jnp.float32),
            pltpu.VMEM_SHARED((CAP,), jnp.float32),
            pltpu.VMEM_SHARED((2 * NT * 16,), jnp.int32),
            pltpu.VMEM_SHARED((NT * SLOT,), jnp.int32),
        ],
    )(_body)
    return f(boxes, scores)


def kernel(boxes, scores):
    out = _nms(boxes.reshape(-1), scores)
    return out[:500].reshape(K_OUT, 5)
